# baseline (speedup 1.0000x reference)
import functools

import jax
import jax.numpy as jnp
from jax import lax
from jax.experimental import pallas as pl
from jax.experimental.pallas import tpu as pltpu
from jax.experimental.pallas import tpu_sc as plsc

H = 1024
EPS = 1e-12
L = 16
NC = 2
NS = 16
NW = NC * NS
HV = H // L
PB = 8
NB = 3


def _rsqrt_vec(x):
    i = plsc.bitcast(x, jnp.int32)
    i = jnp.full((L,), 0x5F3759DF, jnp.int32) - lax.shift_right_arithmetic(i, 1)
    y = plsc.bitcast(i, jnp.float32)
    for _ in range(3):
        y = y * (1.5 - 0.5 * x * y * y)
    return y


def _sc_embed_ln(ids_flat, token_table, pos_table, gamma, beta, *, nb, s_len):
    ppw = s_len // NW
    nch = ppw // PB
    cr = nb * PB
    nblk = cr // L
    bpb = L // PB

    mesh = plsc.VectorSubcoreMesh(
        core_axis_name="c", subcore_axis_name="s",
        num_cores=NC, num_subcores=NS)

    @functools.partial(
        pl.kernel,
        out_type=jax.ShapeDtypeStruct((nb * s_len, H), jnp.float32),
        mesh=mesh,
        scratch_types=[
            pltpu.VMEM((nb, ppw), jnp.int32),
            pltpu.VMEM((H,), jnp.float32),
            pltpu.VMEM((H,), jnp.float32),
            pltpu.VMEM((NB, cr, H), jnp.float32),
            pltpu.VMEM((NB, PB, H), jnp.float32),
            pltpu.VMEM((L, 4 * L + 1), jnp.float32),
            pltpu.SemaphoreType.DMA((NB,)),
            pltpu.SemaphoreType.DMA((NB,)),
        ],
        compiler_params=pltpu.CompilerParams(needs_layout_passes=False),
    )
    def k(ids_hbm, tok_hbm, pos_hbm, gamma_hbm, beta_hbm, out_hbm,
          idx_v, gamma_v, beta_v, tok_b, pos_b, red_v, gsem, wsem):
        wid = lax.axis_index("s") * NC + lax.axis_index("c")
        pos0 = wid * ppw

        for bi in range(nb):
            pltpu.sync_copy(
                ids_hbm.at[pl.ds(bi * s_len + pos0, ppw)], idx_v.at[bi])
        pltpu.sync_copy(gamma_hbm, gamma_v)
        pltpu.sync_copy(beta_hbm, beta_v)

        def start_chunk(c, slot):
            pltpu.async_copy(
                pos_hbm.at[pl.ds(pos0 + c * PB, PB)], pos_b.at[slot],
                gsem.at[slot])
            for bi in range(nb):
                pltpu.async_copy(
                    tok_hbm.at[idx_v.at[bi, pl.ds(c * PB, PB)]],
                    tok_b.at[slot, pl.ds(bi * PB, PB)],
                    gsem.at[slot])

        def wait_chunk(slot):
            pltpu.make_async_copy(
                tok_hbm.at[pl.ds(0, cr)], tok_b.at[slot],
                gsem.at[slot]).wait()
            pltpu.make_async_copy(
                pos_hbm.at[pl.ds(0, PB)], pos_b.at[slot],
                gsem.at[slot]).wait()

        def start_writeout(c, slot):
            for bi in range(nb):
                pltpu.async_copy(
                    tok_b.at[slot, pl.ds(bi * PB, PB)],
                    out_hbm.at[pl.ds(bi * s_len + pos0 + c * PB, PB)],
                    wsem.at[slot])

        def wait_writeout(slot):
            pltpu.make_async_copy(
                tok_b.at[slot], out_hbm.at[pl.ds(0, cr)],
                wsem.at[slot]).wait()

        start_chunk(0, 0)

        @pl.loop(0, nch)
        def chunk_loop(c):
            slot = lax.rem(c, NB)

            @pl.when(c + 1 < nch)
            def _prefetch():
                sp = lax.rem(c + 1, NB)

                @pl.when(c >= NB - 1)
                def _reuse_guard():
                    wait_writeout(sp)
                start_chunk(c + 1, sp)

            wait_chunk(slot)

            rows = lax.iota(jnp.int32, L)
            zero = jnp.zeros((L,), jnp.float32)
            for kp in range(PB):
                def acc_body(j, carry, kp=kp):
                    s1s, s2s = carry[:nb], carry[nb:]
                    p = pos_b[slot, kp, pl.ds(j * L, L)]
                    n1, n2 = [], []
                    for bi in range(nb):
                        r = bi * PB + kp
                        v = tok_b[slot, r, pl.ds(j * L, L)] + p
                        tok_b[slot, r, pl.ds(j * L, L)] = v
                        n1.append(s1s[bi] + v)
                        n2.append(s2s[bi] + v * v)
                    return (*n1, *n2)

                res = lax.fori_loop(0, HV, acc_body, (zero,) * (2 * nb),
                                    unroll=2)
                for bi in range(nb):
                    r = bi * PB + kp
                    blk, rr = divmod(r, L)
                    col = jnp.full((L,), blk * 2 * L + rr, jnp.int32)
                    plsc.store_scatter(red_v, [rows, col], res[bi])
                    plsc.store_scatter(red_v, [rows, col + L], res[nb + bi])

            for blk in range(nblk):
                c0 = blk * 2 * L
                m1 = red_v[0, pl.ds(c0, L)]
                m2 = red_v[0, pl.ds(c0 + L, L)]
                for i in range(1, L):
                    m1 = m1 + red_v[i, pl.ds(c0, L)]
                    m2 = m2 + red_v[i, pl.ds(c0 + L, L)]

                mean_v = m1 * (1.0 / H)
                var_v = m2 * (1.0 / H) - mean_v * mean_v
                rstd_v = _rsqrt_vec(var_v + EPS)
                d_v = -mean_v * rstd_v

                a_sp = [jnp.full((L,), rstd_v[rr]) for rr in range(L)]
                d_sp = [jnp.full((L,), d_v[rr]) for rr in range(L)]

                def grp_body(j, _, blk=blk, a_sp=a_sp, d_sp=d_sp):
                    g = gamma_v[pl.ds(j * L, L)]
                    bb = beta_v[pl.ds(j * L, L)]
                    for rr in range(L):
                        r = blk * L + rr
                        v = tok_b[slot, r, pl.ds(j * L, L)]
                        t = v * a_sp[rr] + d_sp[rr]
                        tok_b[slot, r, pl.ds(j * L, L)] = t * g + bb
                    return 0

                lax.fori_loop(0, HV, grp_body, 0)

            start_writeout(c, slot)

        for i in range(min(NB, nch)):
            wait_writeout((nch - 1 - i) % NB)

    return k(ids_flat, token_table, pos_table, gamma, beta)


def kernel(input_ids, token_table, pos_table, gamma, beta):
    b, s = input_ids.shape
    ids_flat = input_ids.reshape(b * s).astype(jnp.int32)
    out = _sc_embed_ln(ids_flat, token_table, pos_table, gamma, beta,
                       nb=b, s_len=s)
    return out.reshape(b, s, H)
